# baseline (speedup 1.0000x reference)
import functools

import jax
import jax.numpy as jnp
from jax import lax
from jax.experimental import pallas as pl
from jax.experimental.pallas import tpu as pltpu
from jax.experimental.pallas import tpu_sc as plsc

TOKEN = 32
HID = 768
ROW_W = HID
TOK_W = TOKEN * HID
NW = 32
LANES = 16


def _sc_emb_kernel(bits_hbm, table_hbm, out_hbm,
                   tab_v, bits_v, offs_v, buf0, buf1, sem0, sem1):
    wid = lax.axis_index("s") * 2 + lax.axis_index("c")
    n_tok_total = bits_hbm.shape[0] // TOKEN
    tok_pw = n_tok_total // NW
    rows_pw = tok_pw * TOKEN
    tok0 = wid * tok_pw

    pltpu.sync_copy(table_hbm, tab_v.at[pl.ds(TOKEN * HID, TOKEN * HID)])
    pltpu.sync_copy(bits_hbm.at[pl.ds(tok0 * TOKEN, rows_pw)], bits_v)

    def neg_body(i, _):
        o = i * LANES
        tab_v[pl.ds(o, LANES)] = -tab_v[pl.ds(TOKEN * HID + o, LANES)]
        return 0
    lax.fori_loop(0, (TOKEN * HID) // LANES, neg_body, 0)

    lane = lax.broadcasted_iota(jnp.int32, (LANES,), 0)

    def off_body(k, _):
        b16 = bits_v[pl.ds(k * LANES, LANES)]
        t16 = lane + (k % 2) * LANES
        off = t16 * ROW_W + jnp.where(b16 > 0.5, TOKEN * HID, 0)
        offs_v[pl.ds(k * LANES, LANES)] = off
        return 0
    lax.fori_loop(0, rows_pw // LANES, off_body, 0)

    def build_token(t_local, buf):
        def row_body(j, _):
            src = offs_v[pl.ds(t_local * TOKEN + j, LANES)][0]
            dst = j * ROW_W
            for h in range(0, ROW_W, LANES):
                buf[pl.ds(dst + h, LANES)] = tab_v[pl.ds(src + h, LANES)]
            return 0
        lax.fori_loop(0, TOKEN, row_body, 0)

    def out_slice(t_local):
        return out_hbm.at[pl.ds((tok0 + t_local) * TOK_W, TOK_W)]

    n_pairs = tok_pw // 2

    def pair_body(p, _):
        @pl.when(p > 0)
        def _():
            pltpu.make_async_copy(buf0, out_slice(2 * p - 2), sem0).wait()
        build_token(2 * p, buf0)
        pltpu.async_copy(buf0, out_slice(2 * p), sem0)

        @pl.when(p > 0)
        def _():
            pltpu.make_async_copy(buf1, out_slice(2 * p - 1), sem1).wait()
        build_token(2 * p + 1, buf1)
        pltpu.async_copy(buf1, out_slice(2 * p + 1), sem1)
        return 0

    lax.fori_loop(0, n_pairs, pair_body, 0)
    pltpu.make_async_copy(buf0, out_slice(tok_pw - 2), sem0).wait()
    pltpu.make_async_copy(buf1, out_slice(tok_pw - 1), sem1).wait()


def _tc_logit_kernel(bits_ref, table_ref, logit_ref):
    amp = bits_ref[...] * 2.0 - 1.0
    rowsum = jnp.sum(table_ref[...], axis=1)
    logit_ref[...] = amp * rowsum[None, :]


def kernel(text_batch, table):
    B, flat = text_batch.shape
    S = flat // TOKEN
    N = B * S
    bits_flat = text_batch.reshape(N * TOKEN)

    mesh = plsc.VectorSubcoreMesh(core_axis_name="c", subcore_axis_name="s")
    sc_call = functools.partial(
        pl.kernel,
        mesh=mesh,
        out_type=jax.ShapeDtypeStruct((N * TOKEN * HID,), jnp.float32),
        scratch_types=[
            pltpu.VMEM((2 * TOKEN * HID,), jnp.float32),
            pltpu.VMEM((N * TOKEN // NW,), jnp.float32),
            pltpu.VMEM((N * TOKEN // NW + LANES,), jnp.int32),
            pltpu.VMEM((TOK_W,), jnp.float32),
            pltpu.VMEM((TOK_W,), jnp.float32),
            pltpu.SemaphoreType.DMA,
            pltpu.SemaphoreType.DMA,
        ],
    )(_sc_emb_kernel)
    emb = sc_call(bits_flat, table.reshape(TOKEN * HID))

    logit = pl.pallas_call(
        _tc_logit_kernel,
        grid=(1,),
        in_specs=[
            pl.BlockSpec((N, TOKEN), lambda i: (0, 0)),
            pl.BlockSpec((TOKEN, HID), lambda i: (0, 0)),
        ],
        out_specs=pl.BlockSpec((N, TOKEN), lambda i: (0, 0)),
        out_shape=jax.ShapeDtypeStruct((N, TOKEN), jnp.float32),
    )(text_batch.reshape(N, TOKEN), table)

    return emb.reshape(B, S, TOKEN, HID), logit.reshape(B, S, TOKEN, 1)


# speedup vs baseline: 1.6981x; 1.6981x over previous
import functools

import jax
import jax.numpy as jnp
from jax import lax
from jax.experimental import pallas as pl
from jax.experimental.pallas import tpu as pltpu
from jax.experimental.pallas import tpu_sc as plsc

TOKEN = 32
HID = 768
NW = 32
LANES = 16
KROWS = 64


def _sc_emb_kernel(bits_hbm, table2_hbm, out_hbm,
                   bits_v, idx_v, buf0, buf1, gsem0, gsem1, wsem0, wsem1):
    wid = lax.axis_index("s") * 2 + lax.axis_index("c")
    n_rows = bits_hbm.shape[0]
    rows_pw = n_rows // NW
    row0 = wid * rows_pw

    pltpu.sync_copy(bits_hbm.at[pl.ds(row0, rows_pw)], bits_v)

    lane = lax.broadcasted_iota(jnp.int32, (LANES,), 0)

    def idx_body(k, _):
        b16 = bits_v[pl.ds(k * LANES, LANES)]
        t16 = lane + (k % 2) * LANES
        idx_v[pl.ds(k * LANES, LANES)] = t16 + jnp.where(b16 > 0.5, TOKEN, 0)
        return 0
    lax.fori_loop(0, rows_pw // LANES, idx_body, 0)

    n_chunks = rows_pw // KROWS

    def gather(c, buf, gsem):
        pltpu.async_copy(table2_hbm.at[idx_v.at[pl.ds(c * KROWS, KROWS)]],
                         buf, gsem)

    def out_slice(c):
        return out_hbm.at[pl.ds(row0 + c * KROWS, KROWS)]

    def pair_body(p, _):
        @pl.when(p > 0)
        def _():
            pltpu.make_async_copy(buf0, out_slice(2 * p - 2), wsem0).wait()
            pltpu.make_async_copy(buf1, out_slice(2 * p - 1), wsem1).wait()
        gather(2 * p, buf0, gsem0)
        gather(2 * p + 1, buf1, gsem1)
        pltpu.make_async_copy(table2_hbm.at[idx_v.at[pl.ds(0, KROWS)]],
                              buf0, gsem0).wait()
        pltpu.async_copy(buf0, out_slice(2 * p), wsem0)
        pltpu.make_async_copy(table2_hbm.at[idx_v.at[pl.ds(0, KROWS)]],
                              buf1, gsem1).wait()
        pltpu.async_copy(buf1, out_slice(2 * p + 1), wsem1)
        return 0

    lax.fori_loop(0, n_chunks // 2, pair_body, 0)
    pltpu.make_async_copy(buf0, out_slice(n_chunks - 2), wsem0).wait()
    pltpu.make_async_copy(buf1, out_slice(n_chunks - 1), wsem1).wait()


def _tc_logit_kernel(bits_ref, table_ref, logit_ref):
    amp = bits_ref[...] * 2.0 - 1.0
    rowsum = jnp.sum(table_ref[...], axis=1)
    logit_ref[...] = amp * rowsum[None, :]


def kernel(text_batch, table):
    B, flat = text_batch.shape
    S = flat // TOKEN
    N = B * S
    bits_flat = text_batch.reshape(N * TOKEN)
    table2 = jnp.concatenate([-table, table], axis=0)

    mesh = plsc.VectorSubcoreMesh(core_axis_name="c", subcore_axis_name="s")
    sc_call = functools.partial(
        pl.kernel,
        mesh=mesh,
        out_type=jax.ShapeDtypeStruct((N * TOKEN, HID), jnp.float32),
        scratch_types=[
            pltpu.VMEM((N * TOKEN // NW,), jnp.float32),
            pltpu.VMEM((N * TOKEN // NW,), jnp.int32),
            pltpu.VMEM((KROWS, HID), jnp.float32),
            pltpu.VMEM((KROWS, HID), jnp.float32),
            pltpu.SemaphoreType.DMA,
            pltpu.SemaphoreType.DMA,
            pltpu.SemaphoreType.DMA,
            pltpu.SemaphoreType.DMA,
        ],
    )(_sc_emb_kernel)
    emb = sc_call(bits_flat, table2)

    logit = pl.pallas_call(
        _tc_logit_kernel,
        grid=(1,),
        in_specs=[
            pl.BlockSpec((N, TOKEN), lambda i: (0, 0)),
            pl.BlockSpec((TOKEN, HID), lambda i: (0, 0)),
        ],
        out_specs=pl.BlockSpec((N, TOKEN), lambda i: (0, 0)),
        out_shape=jax.ShapeDtypeStruct((N, TOKEN), jnp.float32),
    )(text_batch.reshape(N, TOKEN), table)

    return emb.reshape(B, S, TOKEN, HID), logit.reshape(B, S, TOKEN, 1)


# speedup vs baseline: 6.9843x; 4.1130x over previous
import functools

import jax
import jax.numpy as jnp
from jax import lax
from jax.experimental import pallas as pl
from jax.experimental.pallas import tpu as pltpu
from jax.experimental.pallas import tpu_sc as plsc

TOKEN = 32
HID = 768
NW = 32
LANES = 16
SBLK = 64


def _tc_emb_kernel(bits_ref, table_ref, emb_ref):
    amp = bits_ref[...] * 2.0 - 1.0
    emb_ref[...] = amp[:, :, None] * table_ref[...][None, :, :]


def _sc_logit_kernel(bits_hbm, tableT_hbm, logit_hbm, tab_v, bits_v, logit_v):
    wid = lax.axis_index("s") * 2 + lax.axis_index("c")
    n_vals = bits_hbm.shape[0]
    vals_pw = n_vals // NW
    v0 = wid * vals_pw

    pltpu.sync_copy(tableT_hbm, tab_v)
    pltpu.sync_copy(bits_hbm.at[pl.ds(v0, vals_pw)], bits_v)

    def rsum_body(h, accs):
        acc_lo, acc_hi = accs
        acc_lo = acc_lo + tab_v[pl.ds(h * TOKEN, LANES)]
        acc_hi = acc_hi + tab_v[pl.ds(h * TOKEN + LANES, LANES)]
        return acc_lo, acc_hi

    zeros = jnp.zeros((LANES,), jnp.float32)
    rs_lo, rs_hi = lax.fori_loop(0, HID, rsum_body, (zeros, zeros))

    def logit_body(k, _):
        o = k * TOKEN
        b_lo = bits_v[pl.ds(o, LANES)]
        logit_v[pl.ds(o, LANES)] = (b_lo * 2.0 - 1.0) * rs_lo
        b_hi = bits_v[pl.ds(o + LANES, LANES)]
        logit_v[pl.ds(o + LANES, LANES)] = (b_hi * 2.0 - 1.0) * rs_hi
        return 0

    lax.fori_loop(0, vals_pw // TOKEN, logit_body, 0)
    pltpu.sync_copy(logit_v, logit_hbm.at[pl.ds(v0, vals_pw)])


def kernel(text_batch, table):
    B, flat = text_batch.shape
    S = flat // TOKEN
    N = B * S
    bits = text_batch.reshape(N, TOKEN)

    mesh = plsc.VectorSubcoreMesh(core_axis_name="c", subcore_axis_name="s")
    sc_call = functools.partial(
        pl.kernel,
        mesh=mesh,
        out_type=jax.ShapeDtypeStruct((N * TOKEN,), jnp.float32),
        scratch_types=[
            pltpu.VMEM((TOKEN * HID,), jnp.float32),
            pltpu.VMEM((N * TOKEN // NW,), jnp.float32),
            pltpu.VMEM((N * TOKEN // NW,), jnp.float32),
        ],
    )(_sc_logit_kernel)
    logit = sc_call(text_batch.reshape(N * TOKEN), table.T.reshape(TOKEN * HID))

    emb = pl.pallas_call(
        _tc_emb_kernel,
        grid=(N // SBLK,),
        in_specs=[
            pl.BlockSpec((SBLK, TOKEN), lambda i: (i, 0)),
            pl.BlockSpec((TOKEN, HID), lambda i: (0, 0)),
        ],
        out_specs=pl.BlockSpec((SBLK, TOKEN, HID), lambda i: (i, 0, 0)),
        out_shape=jax.ShapeDtypeStruct((N, TOKEN, HID), jnp.float32),
    )(bits, table)

    return emb.reshape(B, S, TOKEN, HID), logit.reshape(B, S, TOKEN, 1)
